# baseline (speedup 1.0000x reference)
import functools

import jax
import jax.numpy as jnp
from jax import lax
from jax.experimental import pallas as pl
from jax.experimental.pallas import tpu as pltpu
from jax.experimental.pallas import tpu_sc as plsc

N = 50000
NU = 25000
E = 800000
D = 64
NC, NS = 2, 16
NW = NC * NS
ROW = 128
RPT = 200
NROWS = NW * RPT
E_PAD = NROWS * ROW
RPS = NROWS // NS
NACC = 50176
NUACC = 25088
H = 32
K = 8
CH = 40

f32 = jnp.float32
i32 = jnp.int32

_mesh = plsc.VectorSubcoreMesh(core_axis_name="c", subcore_axis_name="s")
_cp = pltpu.CompilerParams(use_tc_tiling_on_sc=False)
_f1 = jax.ShapeDtypeStruct((NC * NACC,), f32)


def _pipe(tref, sbuf, dbuf, acc, vbuf, sem_g, sem_s, nb, r0=0):
    def fire_g(b, p):
        for k_ in range(K):
            pltpu.async_copy(tref.at[sbuf.at[r0 + b * K + k_]], vbuf.at[p, k_], sem_g)

    def wait_g():
        for k_ in range(K):
            pltpu.make_async_copy(tref.at[sbuf.at[r0]], vbuf.at[0, k_], sem_g).wait()

    def fire_s(b, p):
        for k_ in range(K):
            pltpu.async_copy(vbuf.at[p, k_], acc.at[dbuf.at[r0 + b * K + k_]],
                             sem_s, add=True)

    def wait_s():
        for k_ in range(K):
            pltpu.make_async_copy(vbuf.at[0, k_], acc.at[dbuf.at[r0]], sem_s).wait()

    fire_g(0, 0)

    def loop(b, _):
        p = lax.rem(b, 2)
        wait_g()
        pl.when(b >= 1)(wait_s)
        pl.when(b < nb - 1)(lambda: fire_g(b + 1, 1 - p))
        fire_s(b, p)
        return 0

    lax.fori_loop(0, nb, loop, 0)
    wait_s()


@functools.partial(
    pl.kernel,
    out_type=(_f1, _f1, _f1),
    mesh=_mesh,
    compiler_params=_cp,
    scratch_types=[
        pltpu.VMEM_SHARED((NACC,), f32),
        pltpu.VMEM_SHARED((NACC,), f32),
        pltpu.VMEM_SHARED((NACC,), f32),
        pltpu.VMEM((RPT, ROW), i32),
        pltpu.VMEM((ROW,), f32),
        pltpu.VMEM((NACC // NS,), f32),
        pltpu.SemaphoreType.DMA,
    ],
)
def _deg_sc(du, dv, dd, zer1, ou, ov, od, au, av, ad, dbuf, ones, stage, sem):
    c, s = lax.axis_index("c"), lax.axis_index("s")
    w = c * NS + s
    seg = NACC // NS
    sl = pl.ds(s * seg, seg)
    for j in range(ROW // 16):
        ones[pl.ds(j * 16, 16)] = jnp.ones((16,), f32)
    pltpu.sync_copy(zer1.at[sl], stage)
    for acc in (au, av, ad):
        pltpu.sync_copy(stage, acc.at[sl])
    plsc.subcore_barrier()
    row0 = w * RPT
    for dref, acc in ((du, au), (dv, av), (dd, ad)):
        pltpu.sync_copy(dref.at[pl.ds(row0, RPT)], dbuf)

        def body(r, _, acc=acc):
            pl.when(r >= 8)(
                lambda: pltpu.make_async_copy(ones, acc.at[dbuf.at[0]], sem).wait())
            pltpu.async_copy(ones, acc.at[dbuf.at[r]], sem, add=True)
            return 0

        lax.fori_loop(0, RPT, body, 0)
        for _ in range(8):
            pltpu.make_async_copy(ones, acc.at[dbuf.at[0]], sem).wait()
    plsc.subcore_barrier()
    osl = pl.ds(c * NACC + s * seg, seg)
    for out, acc in ((ou, au), (ov, av), (od, ad)):
        pltpu.sync_copy(acc.at[sl], stage)
        pltpu.sync_copy(stage, out.at[osl])


@functools.partial(
    pl.kernel,
    out_type=(_f1, _f1),
    mesh=_mesh,
    compiler_params=_cp,
    scratch_types=[
        pltpu.VMEM_SHARED((NACC,), f32),
        pltpu.VMEM_SHARED((NACC,), f32),
        pltpu.VMEM((RPT, ROW), i32),
        pltpu.VMEM((RPT, ROW), i32),
        pltpu.VMEM((2, K, ROW), f32),
        pltpu.VMEM((NACC // NS,), f32),
        pltpu.SemaphoreType.DMA,
        pltpu.SemaphoreType.DMA,
    ],
)
def _s01_sc(su, du, sv, dv, yu, yv, zer1, ou, ov, au, av, sbuf, dbuf, vbuf,
            stage, sem_g, sem_s):
    c, s = lax.axis_index("c"), lax.axis_index("s")
    w = c * NS + s
    seg = NACC // NS
    sl = pl.ds(s * seg, seg)
    pltpu.sync_copy(zer1.at[sl], stage)
    for acc in (au, av):
        pltpu.sync_copy(stage, acc.at[sl])
    plsc.subcore_barrier()
    row0 = w * RPT
    for sref, dref, tref, acc in ((su, du, yu, au), (sv, dv, yv, av)):
        pltpu.sync_copy(sref.at[pl.ds(row0, RPT)], sbuf)
        pltpu.sync_copy(dref.at[pl.ds(row0, RPT)], dbuf)
        _pipe(tref, sbuf, dbuf, acc, vbuf, sem_g, sem_s, RPT // K)
    plsc.subcore_barrier()
    osl = pl.ds(c * NACC + s * seg, seg)
    for out, acc in ((ou, au), (ov, av)):
        pltpu.sync_copy(acc.at[sl], stage)
        pltpu.sync_copy(stage, out.at[osl])


@functools.partial(
    pl.kernel,
    out_type=jax.ShapeDtypeStruct((NC, NACC, 8), f32),
    mesh=_mesh,
    compiler_params=_cp,
    scratch_types=[
        pltpu.VMEM_SHARED((NACC, 8), f32),
        pltpu.VMEM((RPT, ROW), i32),
        pltpu.VMEM((RPT, ROW), i32),
        pltpu.VMEM((2, K, ROW, 8), f32),
        pltpu.VMEM((NACC // NS, 8), f32),
        pltpu.SemaphoreType.DMA,
        pltpu.SemaphoreType.DMA,
    ],
)
def _uv0_sc(ss, dd, t4, zer4, out, acc, sbuf, dbuf, vbuf, stage, sem_g, sem_s):
    c, s = lax.axis_index("c"), lax.axis_index("s")
    w = c * NS + s
    seg = NACC // NS
    sl = pl.ds(s * seg, seg)
    pltpu.sync_copy(zer4.at[sl], stage)
    pltpu.sync_copy(stage, acc.at[sl])
    plsc.subcore_barrier()
    row0 = w * RPT
    pltpu.sync_copy(ss.at[pl.ds(row0, RPT)], sbuf)
    pltpu.sync_copy(dd.at[pl.ds(row0, RPT)], dbuf)
    _pipe(t4, sbuf, dbuf, acc, vbuf, sem_g, sem_s, RPT // K)
    plsc.subcore_barrier()
    pltpu.sync_copy(acc.at[sl], stage)
    pltpu.sync_copy(stage, out.at[c, sl])


def _wide_pass2(width, acc_len):
    @functools.partial(
        pl.kernel,
        out_type=jax.ShapeDtypeStruct((NC, 2, acc_len, width), f32),
        mesh=_mesh,
        compiler_params=_cp,
        scratch_types=[
            pltpu.VMEM_SHARED((acc_len, width), f32),
            pltpu.VMEM((CH, ROW), i32),
            pltpu.VMEM((CH, ROW), i32),
            pltpu.VMEM((2, K, ROW, width), f32),
            pltpu.VMEM((98, width), f32),
            pltpu.SemaphoreType.DMA,
            pltpu.SemaphoreType.DMA,
        ],
    )
    def _k(s1, d1, s2, d2, t1, t2, zer, out, acc, sbuf, dbuf, vbuf, stage,
           sem_g, sem_s):
        c, s = lax.axis_index("c"), lax.axis_index("s")
        seg = acc_len // NS

        def zero():
            pltpu.sync_copy(zer.at[pl.ds(0, 98)], stage)
            for k_ in range(seg // 98):
                pltpu.sync_copy(stage, acc.at[pl.ds(s * seg + k_ * 98, 98)])

        def drain(j):
            for k_ in range(seg // 98):
                ksl = pl.ds(s * seg + k_ * 98, 98)
                pltpu.sync_copy(acc.at[ksl], stage)
                pltpu.sync_copy(stage, out.at[c, j, ksl])

        def run(sr, dr, tab):
            row0 = s * RPS
            tc_ = tab.at[c]

            def chunk(g, _):
                c0 = row0 + g * CH
                pltpu.sync_copy(sr.at[pl.ds(c0, CH)], sbuf)
                pltpu.sync_copy(dr.at[pl.ds(c0, CH)], dbuf)
                _pipe(tc_, sbuf, dbuf, acc, vbuf, sem_g, sem_s, CH // K)
                return 0

            lax.fori_loop(0, RPS // CH, chunk, 0)

        zero()
        plsc.subcore_barrier()
        run(s1, d1, t1)
        plsc.subcore_barrier()
        drain(0)
        zero()
        plsc.subcore_barrier()
        run(s2, d2, t2)
        plsc.subcore_barrier()
        drain(1)

    return _k


_p32_sc = _wide_pass2(H, NUACC)
_q16_sc = _wide_pass2(16, NACC)


def _mm_relu_body(m_ref, k_ref, b_ref, o_ref, *, relu):
    acc = jnp.dot(m_ref[...], k_ref[...], preferred_element_type=jnp.float32)
    acc = acc + b_ref[...]
    if relu:
        acc = jnp.maximum(acc, 0.0)
    o_ref[...] = acc


def _mm_bias(m, k, b, relu):
    n, c = m.shape
    blk = 5000
    assert n % blk == 0
    return pl.pallas_call(
        functools.partial(_mm_relu_body, relu=relu),
        grid=(n // blk,),
        in_specs=[
            pl.BlockSpec((blk, c), lambda i: (i, 0)),
            pl.BlockSpec((c, D), lambda i: (0, 0)),
            pl.BlockSpec((1, D), lambda i: (0, 0)),
        ],
        out_specs=pl.BlockSpec((blk, D), lambda i: (i, 0)),
        out_shape=jax.ShapeDtypeStruct((n, D), jnp.float32),
    )(m, k, b.reshape(1, D))


def _pad_idx(a, fill):
    a = a.astype(i32)
    return jnp.concatenate([a, fill]).reshape(NROWS, ROW)


def _split(t, width):
    return jnp.transpose(t.reshape(N, D // width, width), (1, 0, 2))


def _two(flat):
    return flat[:N] + flat[NACC:NACC + N]


def kernel(x, edge_index, edge_index_u, edge_index_v, W_u0, b_u0, W_v0, b_v0,
           W_uv0, b_uv0, W_u1, b_u1, W_v1, b_v1, W_uv1, b_uv1):
    x0 = x[:, 0]
    su, du = edge_index_u[0].astype(i32), edge_index_u[1].astype(i32)
    sv, dv = edge_index_v[0].astype(i32), edge_index_v[1].astype(i32)
    ss, dd = edge_index[0].astype(i32), edge_index[1].astype(i32)

    padN = N + (jnp.arange(E_PAD - E, dtype=i32) % (NACC - N))
    padU = NU + (jnp.arange(E_PAD - E, dtype=i32) % (NUACC - NU))
    pad0 = jnp.zeros((E_PAD - E,), i32)
    su_p, du_p = _pad_idx(su, pad0), _pad_idx(du, padN)
    sv_p, dv_p = _pad_idx(sv, pad0), _pad_idx(dv, padN)
    ss_p, dd_p = _pad_idx(ss, pad0), _pad_idx(dd, padN)
    spread = NU + (jnp.arange(E, dtype=i32) % (NUACC - NU))
    duq_p = _pad_idx(jnp.where(du < NU, du, spread), padU)
    dvq_p = _pad_idx(jnp.where(dv >= NU, dv - NU, spread), padU)

    zer1 = jnp.zeros((NACC,), f32)
    zer4 = jnp.zeros((NACC, 8), f32)
    zerh = jnp.zeros((NUACC, H), f32)
    zerq = jnp.zeros((NACC, 16), f32)

    pu, pv, pd = _deg_sc(du_p, dv_p, dd_p, zer1)
    deg_u, deg_v, deg = 1.0 + _two(pu), 1.0 + _two(pv), 1.0 + _two(pd)
    dis_u, dis_v, dis = deg_u ** -0.5, deg_v ** -0.5, deg ** -0.5

    yu = x0 * dis_u
    yv = x0 * dis_v
    ou, ov = _s01_sc(su_p, du_p, sv_p, dv_p, yu, yv, zer1)
    s_u = dis_u * _two(ou) + x0 / deg_u
    s_v = dis_v * _two(ov) + x0 / deg_v
    s_cat = jnp.concatenate([s_u[:NU], s_v[NU:]])

    z = s_cat * dis
    part_u = jnp.arange(N) < NU
    zero_n = jnp.zeros((N,), f32)
    t4 = jnp.stack([
        jnp.where(part_u, z, 0.0),
        jnp.where(part_u, dis, 0.0),
        jnp.where(part_u, 0.0, z),
        jnp.where(part_u, 0.0, dis),
        zero_n, zero_n, zero_n, zero_n,
    ], axis=1)
    a4p = _uv0_sc(ss_p, dd_p, t4, zer4)
    a4 = a4p[0, :N, :4] + a4p[1, :N, :4]

    inv_deg = 1.0 / deg
    selfu = jnp.where(part_u, s_cat * inv_deg, 0.0)
    selfc = jnp.where(part_u, inv_deg, 0.0)
    M = dis[:, None] * a4 + jnp.stack(
        [selfu, selfc, s_cat * inv_deg - selfu, inv_deg - selfc], axis=1)
    Kmat = jnp.concatenate([
        W_u0 @ W_uv0, (b_u0 @ W_uv0)[None, :],
        W_v0 @ W_uv0, (b_v0 @ W_uv0)[None, :],
    ], axis=0)
    x1 = _mm_bias(M, Kmat, b_uv0, relu=True)

    yu_h = _split(x1 * dis_u[:, None], H)
    yv_h = _split(x1 * dis_v[:, None], H)
    pp = _p32_sc(su_p, duq_p, sv_p, dvq_p, yu_h, yv_h, zerh)
    P_u = jnp.concatenate([pp[0, 0, :NU], pp[1, 0, :NU]], axis=1)
    P_v = jnp.concatenate([pp[0, 1, :NU], pp[1, 1, :NU]], axis=1)
    P_u = dis_u[:NU, None] * P_u + x1[:NU] / deg_u[:NU, None]
    P_v = dis_v[NU:, None] * P_v + x1[NU:] / deg_v[NU:, None]
    x2_u = _mm_bias(P_u, W_u1, b_u1, relu=False)
    x2_v = _mm_bias(P_v, W_v1, b_v1, relu=False)
    x2 = jnp.concatenate([x2_u, x2_v], axis=0)

    y2q = _split(x2 * dis[:, None], 16)
    qq = _q16_sc(ss_p, dd_p, ss_p, dd_p, y2q[0:2], y2q[2:4], zerq)
    Q = jnp.concatenate([qq[0, 0, :N], qq[1, 0, :N],
                         qq[0, 1, :N], qq[1, 1, :N]], axis=1)
    Q = dis[:, None] * Q + x2 * inv_deg[:, None]
    return _mm_bias(Q, W_uv1, b_uv1, relu=False)


# speedup vs baseline: 1.0499x; 1.0499x over previous
import functools

import jax
import jax.numpy as jnp
from jax import lax
from jax.experimental import pallas as pl
from jax.experimental.pallas import tpu as pltpu
from jax.experimental.pallas import tpu_sc as plsc

N = 50000
NU = 25000
E = 800000
D = 64
NC, NS = 2, 16
NW = NC * NS
ROW = 128
RPT = 200
NROWS = NW * RPT
E_PAD = NROWS * ROW
RPS = NROWS // NS
NACC = 50176
NUACC = 25088
H = 32
K = 8
CH = 40

f32 = jnp.float32
i32 = jnp.int32

_mesh = plsc.VectorSubcoreMesh(core_axis_name="c", subcore_axis_name="s")
_cp = pltpu.CompilerParams(use_tc_tiling_on_sc=False)
_f1 = jax.ShapeDtypeStruct((NC * NACC,), f32)


def _pipe(tref, sbuf, dbuf, acc, vbuf, sem_g, sem_s, nb, r0=0):
    def fire_g(b, p):
        for k_ in range(K):
            pltpu.async_copy(tref.at[sbuf.at[r0 + b * K + k_]], vbuf.at[p, k_], sem_g)

    def wait_g():
        for k_ in range(K):
            pltpu.make_async_copy(tref.at[sbuf.at[r0]], vbuf.at[0, k_], sem_g).wait()

    def fire_s(b, p):
        for k_ in range(K):
            pltpu.async_copy(vbuf.at[p, k_], acc.at[dbuf.at[r0 + b * K + k_]],
                             sem_s, add=True)

    def wait_s():
        for k_ in range(K):
            pltpu.make_async_copy(vbuf.at[0, k_], acc.at[dbuf.at[r0]], sem_s).wait()

    fire_g(0, 0)

    def loop(b, _):
        p = lax.rem(b, 2)
        wait_g()
        pl.when(b >= 1)(wait_s)
        pl.when(b < nb - 1)(lambda: fire_g(b + 1, 1 - p))
        fire_s(b, p)
        return 0

    lax.fori_loop(0, nb, loop, 0)
    wait_s()


@functools.partial(
    pl.kernel,
    out_type=(_f1, _f1, _f1),
    mesh=_mesh,
    compiler_params=_cp,
    scratch_types=[
        pltpu.VMEM_SHARED((NACC,), f32),
        pltpu.VMEM_SHARED((NACC,), f32),
        pltpu.VMEM_SHARED((NACC,), f32),
        pltpu.VMEM((RPT, ROW), i32),
        pltpu.VMEM((ROW,), f32),
        pltpu.VMEM((NACC // NS,), f32),
        pltpu.SemaphoreType.DMA,
    ],
)
def _deg_sc(du, dv, dd, zer1, ou, ov, od, au, av, ad, dbuf, ones, stage, sem):
    c, s = lax.axis_index("c"), lax.axis_index("s")
    w = c * NS + s
    seg = NACC // NS
    sl = pl.ds(s * seg, seg)
    for j in range(ROW // 16):
        ones[pl.ds(j * 16, 16)] = jnp.ones((16,), f32)
    pltpu.sync_copy(zer1.at[sl], stage)
    for acc in (au, av, ad):
        pltpu.sync_copy(stage, acc.at[sl])
    plsc.subcore_barrier()
    row0 = w * RPT
    for dref, acc in ((du, au), (dv, av), (dd, ad)):
        pltpu.sync_copy(dref.at[pl.ds(row0, RPT)], dbuf)

        def body(r, _, acc=acc):
            pl.when(r >= 8)(
                lambda: pltpu.make_async_copy(ones, acc.at[dbuf.at[0]], sem).wait())
            pltpu.async_copy(ones, acc.at[dbuf.at[r]], sem, add=True)
            return 0

        lax.fori_loop(0, RPT, body, 0)
        for _ in range(8):
            pltpu.make_async_copy(ones, acc.at[dbuf.at[0]], sem).wait()
    plsc.subcore_barrier()
    osl = pl.ds(c * NACC + s * seg, seg)
    for out, acc in ((ou, au), (ov, av), (od, ad)):
        pltpu.sync_copy(acc.at[sl], stage)
        pltpu.sync_copy(stage, out.at[osl])


@functools.partial(
    pl.kernel,
    out_type=(_f1, _f1),
    mesh=_mesh,
    compiler_params=_cp,
    scratch_types=[
        pltpu.VMEM_SHARED((NACC,), f32),
        pltpu.VMEM_SHARED((NACC,), f32),
        pltpu.VMEM((RPT, ROW), i32),
        pltpu.VMEM((RPT, ROW), i32),
        pltpu.VMEM((2, K, ROW), f32),
        pltpu.VMEM((NACC // NS,), f32),
        pltpu.SemaphoreType.DMA,
        pltpu.SemaphoreType.DMA,
    ],
)
def _s01_sc(su, du, sv, dv, yu, yv, zer1, ou, ov, au, av, sbuf, dbuf, vbuf,
            stage, sem_g, sem_s):
    c, s = lax.axis_index("c"), lax.axis_index("s")
    w = c * NS + s
    seg = NACC // NS
    sl = pl.ds(s * seg, seg)
    pltpu.sync_copy(zer1.at[sl], stage)
    for acc in (au, av):
        pltpu.sync_copy(stage, acc.at[sl])
    plsc.subcore_barrier()
    row0 = w * RPT
    for sref, dref, tref, acc in ((su, du, yu, au), (sv, dv, yv, av)):
        pltpu.sync_copy(sref.at[pl.ds(row0, RPT)], sbuf)
        pltpu.sync_copy(dref.at[pl.ds(row0, RPT)], dbuf)
        _pipe(tref, sbuf, dbuf, acc, vbuf, sem_g, sem_s, RPT // K)
    plsc.subcore_barrier()
    osl = pl.ds(c * NACC + s * seg, seg)
    for out, acc in ((ou, au), (ov, av)):
        pltpu.sync_copy(acc.at[sl], stage)
        pltpu.sync_copy(stage, out.at[osl])


@functools.partial(
    pl.kernel,
    out_type=jax.ShapeDtypeStruct((NC, NACC, 8), f32),
    mesh=_mesh,
    compiler_params=_cp,
    scratch_types=[
        pltpu.VMEM_SHARED((NACC, 8), f32),
        pltpu.VMEM((RPT, ROW), i32),
        pltpu.VMEM((RPT, ROW), i32),
        pltpu.VMEM((2, K, ROW, 8), f32),
        pltpu.VMEM((NACC // NS, 8), f32),
        pltpu.SemaphoreType.DMA,
        pltpu.SemaphoreType.DMA,
    ],
)
def _uv0_sc(ss, dd, t4, zer4, out, acc, sbuf, dbuf, vbuf, stage, sem_g, sem_s):
    c, s = lax.axis_index("c"), lax.axis_index("s")
    w = c * NS + s
    seg = NACC // NS
    sl = pl.ds(s * seg, seg)
    pltpu.sync_copy(zer4.at[sl], stage)
    pltpu.sync_copy(stage, acc.at[sl])
    plsc.subcore_barrier()
    row0 = w * RPT
    pltpu.sync_copy(ss.at[pl.ds(row0, RPT)], sbuf)
    pltpu.sync_copy(dd.at[pl.ds(row0, RPT)], dbuf)
    _pipe(t4, sbuf, dbuf, acc, vbuf, sem_g, sem_s, RPT // K)
    plsc.subcore_barrier()
    pltpu.sync_copy(acc.at[sl], stage)
    pltpu.sync_copy(stage, out.at[c, sl])


def _wide_pass(width, acc_len):
    @functools.partial(
        pl.kernel,
        out_type=jax.ShapeDtypeStruct((NC, acc_len, width), f32),
        mesh=_mesh,
        compiler_params=_cp,
        scratch_types=[
            pltpu.VMEM_SHARED((acc_len, width), f32),
            pltpu.VMEM((CH, ROW), i32),
            pltpu.VMEM((CH, ROW), i32),
            pltpu.VMEM((2, K, ROW, width), f32),
            pltpu.VMEM((98, width), f32),
            pltpu.SemaphoreType.DMA,
            pltpu.SemaphoreType.DMA,
        ],
    )
    def _k(srows, drows, tab, zer, out, acc, sbuf, dbuf, vbuf, stage, sem_g, sem_s):
        c, s = lax.axis_index("c"), lax.axis_index("s")
        seg = acc_len // NS
        pltpu.sync_copy(zer.at[pl.ds(0, 98)], stage)
        for k_ in range(seg // 98):
            pltpu.sync_copy(stage, acc.at[pl.ds(s * seg + k_ * 98, 98)])
        plsc.subcore_barrier()
        row0 = s * RPS
        tc = tab.at[c]

        def chunk(g, _):
            c0 = row0 + g * CH
            pltpu.sync_copy(srows.at[pl.ds(c0, CH)], sbuf)
            pltpu.sync_copy(drows.at[pl.ds(c0, CH)], dbuf)
            _pipe(tc, sbuf, dbuf, acc, vbuf, sem_g, sem_s, CH // K)
            return 0

        lax.fori_loop(0, RPS // CH, chunk, 0)
        plsc.subcore_barrier()
        for k_ in range(seg // 98):
            ksl = pl.ds(s * seg + k_ * 98, 98)
            pltpu.sync_copy(acc.at[ksl], stage)
            pltpu.sync_copy(stage, out.at[c, ksl])

    return _k


_p32_sc = _wide_pass(H, NUACC)
_q16_sc = _wide_pass(16, NACC)


def _mm_relu_body(m_ref, k_ref, b_ref, o_ref, *, relu):
    acc = jnp.dot(m_ref[...], k_ref[...], preferred_element_type=jnp.float32)
    acc = acc + b_ref[...]
    if relu:
        acc = jnp.maximum(acc, 0.0)
    o_ref[...] = acc


def _mm_bias(m, k, b, relu):
    n, c = m.shape
    blk = 5000
    assert n % blk == 0
    return pl.pallas_call(
        functools.partial(_mm_relu_body, relu=relu),
        grid=(n // blk,),
        in_specs=[
            pl.BlockSpec((blk, c), lambda i: (i, 0)),
            pl.BlockSpec((c, D), lambda i: (0, 0)),
            pl.BlockSpec((1, D), lambda i: (0, 0)),
        ],
        out_specs=pl.BlockSpec((blk, D), lambda i: (i, 0)),
        out_shape=jax.ShapeDtypeStruct((n, D), jnp.float32),
    )(m, k, b.reshape(1, D))


def _pad_idx(a, fill):
    a = a.astype(i32)
    return jnp.concatenate([a, fill]).reshape(NROWS, ROW)


def _split(t, width):
    return jnp.transpose(t.reshape(N, D // width, width), (1, 0, 2))


def _two(flat):
    return flat[:N] + flat[NACC:NACC + N]


def kernel(x, edge_index, edge_index_u, edge_index_v, W_u0, b_u0, W_v0, b_v0,
           W_uv0, b_uv0, W_u1, b_u1, W_v1, b_v1, W_uv1, b_uv1):
    x0 = x[:, 0]
    su, du = edge_index_u[0].astype(i32), edge_index_u[1].astype(i32)
    sv, dv = edge_index_v[0].astype(i32), edge_index_v[1].astype(i32)
    ss, dd = edge_index[0].astype(i32), edge_index[1].astype(i32)

    padN = N + (jnp.arange(E_PAD - E, dtype=i32) % (NACC - N))
    padU = NU + (jnp.arange(E_PAD - E, dtype=i32) % (NUACC - NU))
    pad0 = jnp.zeros((E_PAD - E,), i32)
    su_p, du_p = _pad_idx(su, pad0), _pad_idx(du, padN)
    sv_p, dv_p = _pad_idx(sv, pad0), _pad_idx(dv, padN)
    ss_p, dd_p = _pad_idx(ss, pad0), _pad_idx(dd, padN)
    spread = NU + (jnp.arange(E, dtype=i32) % (NUACC - NU))
    duq_p = _pad_idx(jnp.where(du < NU, du, spread), padU)
    dvq_p = _pad_idx(jnp.where(dv >= NU, dv - NU, spread), padU)

    zer1 = jnp.zeros((NACC,), f32)
    zer4 = jnp.zeros((NACC, 8), f32)
    zerh = jnp.zeros((NUACC, H), f32)
    zerq = jnp.zeros((NACC, 16), f32)

    pu, pv, pd = _deg_sc(du_p, dv_p, dd_p, zer1)
    deg_u, deg_v, deg = 1.0 + _two(pu), 1.0 + _two(pv), 1.0 + _two(pd)
    dis_u, dis_v, dis = deg_u ** -0.5, deg_v ** -0.5, deg ** -0.5

    yu = x0 * dis_u
    yv = x0 * dis_v
    ou, ov = _s01_sc(su_p, du_p, sv_p, dv_p, yu, yv, zer1)
    s_u = dis_u * _two(ou) + x0 / deg_u
    s_v = dis_v * _two(ov) + x0 / deg_v
    s_cat = jnp.concatenate([s_u[:NU], s_v[NU:]])

    z = s_cat * dis
    part_u = jnp.arange(N) < NU
    zero_n = jnp.zeros((N,), f32)
    t4 = jnp.stack([
        jnp.where(part_u, z, 0.0),
        jnp.where(part_u, dis, 0.0),
        jnp.where(part_u, 0.0, z),
        jnp.where(part_u, 0.0, dis),
        zero_n, zero_n, zero_n, zero_n,
    ], axis=1)
    a4p = _uv0_sc(ss_p, dd_p, t4, zer4)
    a4 = a4p[0, :N, :4] + a4p[1, :N, :4]

    inv_deg = 1.0 / deg
    selfu = jnp.where(part_u, s_cat * inv_deg, 0.0)
    selfc = jnp.where(part_u, inv_deg, 0.0)
    M = dis[:, None] * a4 + jnp.stack(
        [selfu, selfc, s_cat * inv_deg - selfu, inv_deg - selfc], axis=1)
    Kmat = jnp.concatenate([
        W_u0 @ W_uv0, (b_u0 @ W_uv0)[None, :],
        W_v0 @ W_uv0, (b_v0 @ W_uv0)[None, :],
    ], axis=0)
    x1 = _mm_bias(M, Kmat, b_uv0, relu=True)

    yu_h = _split(x1 * dis_u[:, None], H)
    yv_h = _split(x1 * dis_v[:, None], H)
    ppu = _p32_sc(su_p, duq_p, yu_h, zerh)
    ppv = _p32_sc(sv_p, dvq_p, yv_h, zerh)
    P_u = jnp.concatenate([ppu[0, :NU], ppu[1, :NU]], axis=1)
    P_v = jnp.concatenate([ppv[0, :NU], ppv[1, :NU]], axis=1)
    P_u = dis_u[:NU, None] * P_u + x1[:NU] / deg_u[:NU, None]
    P_v = dis_v[NU:, None] * P_v + x1[NU:] / deg_v[NU:, None]
    x2_u = _mm_bias(P_u, W_u1, b_u1, relu=False)
    x2_v = _mm_bias(P_v, W_v1, b_v1, relu=False)
    x2 = jnp.concatenate([x2_u, x2_v], axis=0)

    y2q = _split(x2 * dis[:, None], 16)
    qa = _q16_sc(ss_p, dd_p, y2q[0:2], zerq)
    qb = _q16_sc(ss_p, dd_p, y2q[2:4], zerq)
    Q = jnp.concatenate([qa[0, :N], qa[1, :N], qb[0, :N], qb[1, :N]], axis=1)
    Q = dis[:, None] * Q + x2 * inv_deg[:, None]
    return _mm_bias(Q, W_uv1, b_uv1, relu=False)
